# baseline (speedup 1.0000x reference)
import functools

import jax
import jax.numpy as jnp
from jax import lax
from jax.experimental import pallas as pl
from jax.experimental.pallas import tpu as pltpu
from jax.experimental.pallas import tpu_sc as plsc

_N = 10000
_E = 160000
_NSC = 2
_NTILE = 16
_BATCH = 96
_NBUF = 3
_AHEAD = 2
_EPT = _E // (_NSC * _NTILE)
_NB = -(-(-(-_EPT // _BATCH)) // (2 * _NBUF)) * (2 * _NBUF)
_EPAD = _NB * _BATCH
_NBLK = (_N + 16 + 127) // 128
_NP = _NBLK * 128
_BN = 1000
_GRID = _N // _BN


def _make_agg(C: int, W: int):
    mesh = plsc.VectorSubcoreMesh(core_axis_name="c", subcore_axis_name="s")

    @functools.partial(
        pl.kernel,
        out_type=jax.ShapeDtypeStruct((_NSC, C, _NP, W), jnp.float32),
        mesh=mesh,
        scratch_types=[
            pltpu.VMEM((_NB // 2, _BATCH), jnp.int32),
            pltpu.VMEM((_NB // 2, _BATCH), jnp.int32),
            pltpu.VMEM((_NBUF, _BATCH, W), jnp.float32),
            pltpu.VMEM((8, W), jnp.float32),
            pltpu.VMEM_SHARED((_NP, W), jnp.float32),
            [pltpu.SemaphoreType.DMA] * _NBUF,
            [pltpu.SemaphoreType.DMA] * _NBUF,
        ],
    )
    def agg(g_hbm, srcb_hbm, dstb_hbm, zeros_hbm, out_hbm,
            src_v, dst_v, rows_v, zero_v, acc, gsem, ssem):
        cid = lax.axis_index("c")
        sid = lax.axis_index("s")
        t = cid * _NTILE + sid
        nbp = _NB // 2
        pltpu.sync_copy(zeros_hbm, zero_v)
        for c in range(C):
            for k in range((_NBLK + _NTILE - 1) // _NTILE):
                blk = sid + _NTILE * k
                @pl.when(blk < _NBLK)
                def _():
                    for z in range(16):
                        pltpu.sync_copy(
                            zero_v, acc.at[pl.ds(blk * 128 + z * 8, 8)])
            plsc.subcore_barrier()

            for ph in range(2):
                pltpu.sync_copy(srcb_hbm.at[c, t, ph], src_v)
                pltpu.sync_copy(dstb_hbm.at[t, ph], dst_v)
                for b in range(_AHEAD):
                    pltpu.async_copy(g_hbm.at[src_v.at[b]], rows_v.at[b],
                                     gsem[b])

                def body(g, carry):
                    for b in range(_NBUF):
                        j = g * _NBUF + b
                        pltpu.make_async_copy(
                            g_hbm.at[src_v.at[j]], rows_v.at[b],
                            gsem[b]).wait()
                        pltpu.async_copy(rows_v.at[b], acc.at[dst_v.at[j]],
                                         ssem[b], add=True)
                        jn = j + _AHEAD
                        bn = (b + _AHEAD) % _NBUF

                        @pl.when(jn < nbp)
                        def _():
                            @pl.when(jn >= _NBUF)
                            def _():
                                pltpu.make_async_copy(
                                    rows_v.at[bn],
                                    acc.at[dst_v.at[jn - _NBUF]],
                                    ssem[bn]).wait()
                            pltpu.async_copy(g_hbm.at[src_v.at[jn]],
                                             rows_v.at[bn], gsem[bn])
                    return carry

                lax.fori_loop(0, nbp // _NBUF, body, 0)
                for b in range(_NBUF):
                    pltpu.make_async_copy(
                        rows_v.at[b], acc.at[dst_v.at[nbp - _NBUF + b]],
                        ssem[b]).wait()
            plsc.subcore_barrier()
            for k in range((_NBLK + _NTILE - 1) // _NTILE):
                blk = sid + _NTILE * k
                @pl.when(blk < _NBLK)
                def _():
                    pltpu.sync_copy(acc.at[pl.ds(blk * 128, 128)],
                                    out_hbm.at[cid, c, pl.ds(blk * 128, 128)])
            plsc.subcore_barrier()

    return agg


def _make_hist():
    mesh = plsc.VectorSubcoreMesh(core_axis_name="c", subcore_axis_name="s")

    @functools.partial(
        pl.kernel,
        out_type=jax.ShapeDtypeStruct((_NSC, _NP, 128), jnp.float32),
        mesh=mesh,
        scratch_types=[
            pltpu.VMEM((_NB, _BATCH), jnp.int32),
            pltpu.VMEM((_BATCH, 128), jnp.float32),
            pltpu.VMEM((128, 128), jnp.float32),
            pltpu.VMEM_SHARED((_NP, 128), jnp.float32),
        ],
    )
    def hist(onesb_hbm, dstb_hbm, zeros_hbm, out_hbm,
             dst_v, ones_v, zero_v, acc):
        cid = lax.axis_index("c")
        sid = lax.axis_index("s")
        t = cid * _NTILE + sid
        pltpu.sync_copy(onesb_hbm, ones_v)
        pltpu.sync_copy(zeros_hbm, zero_v)
        pltpu.sync_copy(dstb_hbm.at[t], dst_v)
        for k in range((_NBLK + _NTILE - 1) // _NTILE):
            blk = sid + _NTILE * k
            @pl.when(blk < _NBLK)
            def _():
                pltpu.sync_copy(zero_v, acc.at[pl.ds(blk * 128, 128)])
        plsc.subcore_barrier()

        def body(j, carry):
            pltpu.sync_copy(ones_v, acc.at[dst_v.at[j]], add=True)
            return carry

        lax.fori_loop(0, _NB, body, 0)
        plsc.subcore_barrier()
        for k in range((_NBLK + _NTILE - 1) // _NTILE):
            blk = sid + _NTILE * k
            @pl.when(blk < _NBLK)
            def _():
                pltpu.sync_copy(acc.at[pl.ds(blk * 128, 128)],
                                out_hbm.at[cid, pl.ds(blk * 128, 128)])
        plsc.subcore_barrier()

    return hist


_agg_deg = _make_hist()
_agg_2 = _make_agg(2, 128)
_agg_4 = _make_agg(4, 128)
_agg_1 = _make_agg(1, 128)


def _t1_body(dp_ref, x_ref, dinv_ref, gx_ref):
    deg = dp_ref[0, :, 0:1] + dp_ref[1, :, 0:1] + 1.0
    dinv = lax.rsqrt(deg)
    dinv_ref[...] = dinv
    g = x_ref[...] * dinv
    gx_ref[0] = g[:, :128]
    gx_ref[1] = g[:, 128:]


def _t2_body(p_ref, gx_ref, dinv_ref, w1_ref, b1_ref, g1_ref):
    dinv = dinv_ref[...]
    a0 = (p_ref[0, 0] + p_ref[1, 0] + gx_ref[0]) * dinv
    a1 = (p_ref[0, 1] + p_ref[1, 1] + gx_ref[1]) * dinv
    ax = jnp.concatenate([a0, a1], axis=1)
    h = jnp.dot(ax, w1_ref[...], preferred_element_type=jnp.float32)
    h = jnp.maximum(h + b1_ref[...], 0.0)
    g1 = h * dinv
    for c in range(4):
        g1_ref[c] = g1[:, c * 128:(c + 1) * 128]


def _t3_body(q_ref, g1_ref, dinv_ref, w2_ref, b2_ref, w3_ref, g3_ref):
    dinv = dinv_ref[...]
    cols = [(q_ref[0, c] + q_ref[1, c] + g1_ref[c]) * dinv for c in range(4)]
    a = jnp.concatenate(cols, axis=1)
    h = jnp.dot(a, w2_ref[...], preferred_element_type=jnp.float32)
    h = jnp.maximum(h + b2_ref[...], 0.0)
    m3 = jnp.dot(h, w3_ref[...], preferred_element_type=jnp.float32)
    g3_ref[0] = m3 * dinv


def _t4_body(r_ref, g3_ref, dinv_ref, b3_ref, o_ref):
    z = (r_ref[0, 0] + r_ref[1, 0] + g3_ref[0]) * dinv_ref[...] + b3_ref[...]
    m = jnp.max(z, axis=1, keepdims=True)
    e = jnp.exp(z - m)
    o_ref[...] = z - m - jnp.log(jnp.sum(e, axis=1, keepdims=True))


def kernel(x, edge_index, W1, b1, W2, b2, W3, b3):
    src = edge_index[0]
    dst = edge_index[1]
    srcp = jnp.pad(src.reshape(_NSC * _NTILE, _EPT),
                   ((0, 0), (0, _EPAD - _EPT)))
    dstp = jnp.pad(dst.reshape(_NSC * _NTILE, _EPT),
                   ((0, 0), (0, _EPAD - _EPT)), constant_values=_N)
    srcb = (srcp[None] + (jnp.arange(4, dtype=jnp.int32) * _N)[:, None, None])
    srcb = srcb.reshape(4, _NSC * _NTILE, 2, _NB // 2, _BATCH)
    dstb = dstp.reshape(_NSC * _NTILE, 2, _NB // 2, _BATCH)
    dstbh = dstp.reshape(_NSC * _NTILE, _NB, _BATCH)
    ones128 = jnp.ones((_BATCH, 128), jnp.float32)
    z128 = jnp.zeros((128, 128), jnp.float32)
    z8 = jnp.zeros((8, 128), jnp.float32)
    b1r = b1.reshape(1, -1)
    b2r = b2.reshape(1, -1)
    b3r = b3.reshape(1, -1)

    degp = _agg_deg(ones128, dstbh, z128)

    dinv, gx = pl.pallas_call(
        _t1_body,
        grid=(_GRID,),
        in_specs=[
            pl.BlockSpec((2, _BN, 128), lambda i: (0, i, 0)),
            pl.BlockSpec((_BN, 256), lambda i: (i, 0)),
        ],
        out_specs=[
            pl.BlockSpec((_BN, 1), lambda i: (i, 0)),
            pl.BlockSpec((2, _BN, 128), lambda i: (0, i, 0)),
        ],
        out_shape=[
            jax.ShapeDtypeStruct((_N, 1), jnp.float32),
            jax.ShapeDtypeStruct((2, _N, 128), jnp.float32),
        ],
    )(degp, x)

    p1 = _agg_2(gx.reshape(2 * _N, 128), srcb[:2], dstb, z8)

    g1 = pl.pallas_call(
        _t2_body,
        grid=(_GRID,),
        in_specs=[
            pl.BlockSpec((2, 2, _BN, 128), lambda i: (0, 0, i, 0)),
            pl.BlockSpec((2, _BN, 128), lambda i: (0, i, 0)),
            pl.BlockSpec((_BN, 1), lambda i: (i, 0)),
            pl.BlockSpec((256, 512), lambda i: (0, 0)),
            pl.BlockSpec((1, 512), lambda i: (0, 0)),
        ],
        out_specs=pl.BlockSpec((4, _BN, 128), lambda i: (0, i, 0)),
        out_shape=jax.ShapeDtypeStruct((4, _N, 128), jnp.float32),
    )(p1, gx, dinv, W1, b1r)

    p2 = _agg_4(g1.reshape(4 * _N, 128), srcb, dstb, z8)

    g3 = pl.pallas_call(
        _t3_body,
        grid=(_GRID,),
        in_specs=[
            pl.BlockSpec((2, 4, _BN, 128), lambda i: (0, 0, i, 0)),
            pl.BlockSpec((4, _BN, 128), lambda i: (0, i, 0)),
            pl.BlockSpec((_BN, 1), lambda i: (i, 0)),
            pl.BlockSpec((512, 512), lambda i: (0, 0)),
            pl.BlockSpec((1, 512), lambda i: (0, 0)),
            pl.BlockSpec((512, 128), lambda i: (0, 0)),
        ],
        out_specs=pl.BlockSpec((1, _BN, 128), lambda i: (0, i, 0)),
        out_shape=jax.ShapeDtypeStruct((1, _N, 128), jnp.float32),
    )(p2, g1, dinv, W2, b2r, W3)

    p3 = _agg_1(g3.reshape(_N, 128), srcb[:1], dstb, z8)

    out = pl.pallas_call(
        _t4_body,
        grid=(_GRID,),
        in_specs=[
            pl.BlockSpec((2, 1, _BN, 128), lambda i: (0, 0, i, 0)),
            pl.BlockSpec((1, _BN, 128), lambda i: (0, i, 0)),
            pl.BlockSpec((_BN, 1), lambda i: (i, 0)),
            pl.BlockSpec((1, 128), lambda i: (0, 0)),
        ],
        out_specs=pl.BlockSpec((_BN, 128), lambda i: (i, 0)),
        out_shape=jax.ShapeDtypeStruct((_N, 128), jnp.float32),
    )(p3, g3, dinv, b3r)

    return out


# speedup vs baseline: 1.3453x; 1.3453x over previous
import functools

import jax
import jax.numpy as jnp
from jax import lax
from jax.experimental import pallas as pl
from jax.experimental.pallas import tpu as pltpu
from jax.experimental.pallas import tpu_sc as plsc

_N = 10000
_E = 160000
_NSC = 2
_NTILE = 16
_BATCH = 128
_NBUF = 2
_AHEAD = 2
_EPT = _E // (_NSC * _NTILE)
_NB = -(-(-(-_EPT // _BATCH)) // _NBUF) * _NBUF
_EPAD = _NB * _BATCH
_NBLK = (_N + 16 + 127) // 128
_NP = _NBLK * 128
_BN = 1000
_GRID = _N // _BN


def _make_agg(C: int, W: int):
    mesh = plsc.VectorSubcoreMesh(core_axis_name="c", subcore_axis_name="s")

    @functools.partial(
        pl.kernel,
        out_type=jax.ShapeDtypeStruct((_NSC, C, _NP, W), jnp.float32),
        mesh=mesh,
        scratch_types=[
            pltpu.VMEM((_NB, _BATCH), jnp.int32),
            pltpu.VMEM((_NB, _BATCH), jnp.int32),
            pltpu.VMEM((_NBUF, _BATCH, W), jnp.float32),
            pltpu.VMEM((8, W), jnp.float32),
            pltpu.VMEM_SHARED((_NP, W), jnp.float32),
            [pltpu.SemaphoreType.DMA] * _NBUF,
            [pltpu.SemaphoreType.DMA] * _NBUF,
        ],
    )
    def agg(g_hbm, srcb_hbm, dstb_hbm, zeros_hbm, out_hbm,
            src_v, dst_v, rows_v, zero_v, acc, gsem, ssem):
        cid = lax.axis_index("c")
        sid = lax.axis_index("s")
        t = cid * _NTILE + sid
        pltpu.sync_copy(zeros_hbm, zero_v)
        pltpu.sync_copy(dstb_hbm.at[t], dst_v)
        for c in range(C):
            for k in range((_NBLK + _NTILE - 1) // _NTILE):
                blk = sid + _NTILE * k
                @pl.when(blk < _NBLK)
                def _():
                    for z in range(16):
                        pltpu.sync_copy(
                            zero_v, acc.at[pl.ds(blk * 128 + z * 8, 8)])
            pltpu.sync_copy(srcb_hbm.at[c, t], src_v)
            plsc.subcore_barrier()

            for b in range(_AHEAD):
                pltpu.async_copy(g_hbm.at[src_v.at[b]], rows_v.at[b], gsem[b])

            def body(g, carry):
                for b in range(_NBUF):
                    j = g * _NBUF + b
                    pltpu.make_async_copy(
                        g_hbm.at[src_v.at[j]], rows_v.at[b], gsem[b]).wait()
                    pltpu.async_copy(rows_v.at[b], acc.at[dst_v.at[j]],
                                     ssem[b], add=True)
                    jn = j + _AHEAD
                    bn = (b + _AHEAD) % _NBUF

                    @pl.when(jn < _NB)
                    def _():
                        @pl.when(jn >= _NBUF)
                        def _():
                            pltpu.make_async_copy(
                                rows_v.at[bn], acc.at[dst_v.at[jn - _NBUF]],
                                ssem[bn]).wait()
                        pltpu.async_copy(g_hbm.at[src_v.at[jn]],
                                         rows_v.at[bn], gsem[bn])
                return carry

            lax.fori_loop(0, _NB // _NBUF, body, 0)
            for b in range(_NBUF):
                pltpu.make_async_copy(
                    rows_v.at[b], acc.at[dst_v.at[_NB - _NBUF + b]],
                    ssem[b]).wait()
            plsc.subcore_barrier()
            for k in range((_NBLK + _NTILE - 1) // _NTILE):
                blk = sid + _NTILE * k
                @pl.when(blk < _NBLK)
                def _():
                    pltpu.sync_copy(acc.at[pl.ds(blk * 128, 128)],
                                    out_hbm.at[cid, c, pl.ds(blk * 128, 128)])
            plsc.subcore_barrier()

    return agg


def _make_hist():
    mesh = plsc.VectorSubcoreMesh(core_axis_name="c", subcore_axis_name="s")

    @functools.partial(
        pl.kernel,
        out_type=jax.ShapeDtypeStruct((_NSC, _NP, 128), jnp.float32),
        mesh=mesh,
        scratch_types=[
            pltpu.VMEM((_NB, _BATCH), jnp.int32),
            pltpu.VMEM((_BATCH, 128), jnp.float32),
            pltpu.VMEM((128, 128), jnp.float32),
            pltpu.VMEM_SHARED((_NP, 128), jnp.float32),
        ],
    )
    def hist(onesb_hbm, dstb_hbm, zeros_hbm, out_hbm,
             dst_v, ones_v, zero_v, acc):
        cid = lax.axis_index("c")
        sid = lax.axis_index("s")
        t = cid * _NTILE + sid
        pltpu.sync_copy(onesb_hbm, ones_v)
        pltpu.sync_copy(zeros_hbm, zero_v)
        pltpu.sync_copy(dstb_hbm.at[t], dst_v)
        for k in range((_NBLK + _NTILE - 1) // _NTILE):
            blk = sid + _NTILE * k
            @pl.when(blk < _NBLK)
            def _():
                pltpu.sync_copy(zero_v, acc.at[pl.ds(blk * 128, 128)])
        plsc.subcore_barrier()

        def body(j, carry):
            pltpu.sync_copy(ones_v, acc.at[dst_v.at[j]], add=True)
            return carry

        lax.fori_loop(0, _NB, body, 0)
        plsc.subcore_barrier()
        for k in range((_NBLK + _NTILE - 1) // _NTILE):
            blk = sid + _NTILE * k
            @pl.when(blk < _NBLK)
            def _():
                pltpu.sync_copy(acc.at[pl.ds(blk * 128, 128)],
                                out_hbm.at[cid, pl.ds(blk * 128, 128)])
        plsc.subcore_barrier()

    return hist


_agg_deg = _make_hist()
_agg_2 = _make_agg(2, 128)
_agg_4 = _make_agg(4, 128)
_agg_1 = _make_agg(1, 128)


def _t1_body(dp_ref, x_ref, dinv_ref, gx_ref):
    deg = dp_ref[0, :, 0:1] + dp_ref[1, :, 0:1] + 1.0
    dinv = lax.rsqrt(deg)
    dinv_ref[...] = dinv
    g = x_ref[...] * dinv
    gx_ref[0] = g[:, :128]
    gx_ref[1] = g[:, 128:]


def _t2_body(p_ref, gx_ref, dinv_ref, w1_ref, b1_ref, g1_ref):
    dinv = dinv_ref[...]
    a0 = (p_ref[0, 0] + p_ref[1, 0] + gx_ref[0]) * dinv
    a1 = (p_ref[0, 1] + p_ref[1, 1] + gx_ref[1]) * dinv
    ax = jnp.concatenate([a0, a1], axis=1)
    h = jnp.dot(ax, w1_ref[...], preferred_element_type=jnp.float32)
    h = jnp.maximum(h + b1_ref[...], 0.0)
    g1 = h * dinv
    for c in range(4):
        g1_ref[c] = g1[:, c * 128:(c + 1) * 128]


def _t3_body(q_ref, g1_ref, dinv_ref, w2_ref, b2_ref, w3_ref, g3_ref):
    dinv = dinv_ref[...]
    cols = [(q_ref[0, c] + q_ref[1, c] + g1_ref[c]) * dinv for c in range(4)]
    a = jnp.concatenate(cols, axis=1)
    h = jnp.dot(a, w2_ref[...], preferred_element_type=jnp.float32)
    h = jnp.maximum(h + b2_ref[...], 0.0)
    m3 = jnp.dot(h, w3_ref[...], preferred_element_type=jnp.float32)
    g3_ref[0] = m3 * dinv


def _t4_body(r_ref, g3_ref, dinv_ref, b3_ref, o_ref):
    z = (r_ref[0, 0] + r_ref[1, 0] + g3_ref[0]) * dinv_ref[...] + b3_ref[...]
    m = jnp.max(z, axis=1, keepdims=True)
    e = jnp.exp(z - m)
    o_ref[...] = z - m - jnp.log(jnp.sum(e, axis=1, keepdims=True))


def kernel(x, edge_index, W1, b1, W2, b2, W3, b3):
    src = edge_index[0]
    dst = edge_index[1]
    srcp = jnp.pad(src.reshape(_NSC * _NTILE, _EPT),
                   ((0, 0), (0, _EPAD - _EPT)))
    dstp = jnp.pad(dst.reshape(_NSC * _NTILE, _EPT),
                   ((0, 0), (0, _EPAD - _EPT)), constant_values=_N)
    srcb = (srcp[None] + (jnp.arange(4, dtype=jnp.int32) * _N)[:, None, None])
    srcb = srcb.reshape(4, _NSC * _NTILE, _NB, _BATCH)
    dstb = dstp.reshape(_NSC * _NTILE, _NB, _BATCH)
    ones128 = jnp.ones((_BATCH, 128), jnp.float32)
    z128 = jnp.zeros((128, 128), jnp.float32)
    z8 = jnp.zeros((8, 128), jnp.float32)
    b1r = b1.reshape(1, -1)
    b2r = b2.reshape(1, -1)
    b3r = b3.reshape(1, -1)

    degp = _agg_deg(ones128, dstb, z128)

    dinv, gx = pl.pallas_call(
        _t1_body,
        grid=(_GRID,),
        in_specs=[
            pl.BlockSpec((2, _BN, 128), lambda i: (0, i, 0)),
            pl.BlockSpec((_BN, 256), lambda i: (i, 0)),
        ],
        out_specs=[
            pl.BlockSpec((_BN, 1), lambda i: (i, 0)),
            pl.BlockSpec((2, _BN, 128), lambda i: (0, i, 0)),
        ],
        out_shape=[
            jax.ShapeDtypeStruct((_N, 1), jnp.float32),
            jax.ShapeDtypeStruct((2, _N, 128), jnp.float32),
        ],
    )(degp, x)

    p1 = _agg_2(gx.reshape(2 * _N, 128), srcb[:2], dstb, z8)

    g1 = pl.pallas_call(
        _t2_body,
        grid=(_GRID,),
        in_specs=[
            pl.BlockSpec((2, 2, _BN, 128), lambda i: (0, 0, i, 0)),
            pl.BlockSpec((2, _BN, 128), lambda i: (0, i, 0)),
            pl.BlockSpec((_BN, 1), lambda i: (i, 0)),
            pl.BlockSpec((256, 512), lambda i: (0, 0)),
            pl.BlockSpec((1, 512), lambda i: (0, 0)),
        ],
        out_specs=pl.BlockSpec((4, _BN, 128), lambda i: (0, i, 0)),
        out_shape=jax.ShapeDtypeStruct((4, _N, 128), jnp.float32),
    )(p1, gx, dinv, W1, b1r)

    p2 = _agg_4(g1.reshape(4 * _N, 128), srcb, dstb, z8)

    g3 = pl.pallas_call(
        _t3_body,
        grid=(_GRID,),
        in_specs=[
            pl.BlockSpec((2, 4, _BN, 128), lambda i: (0, 0, i, 0)),
            pl.BlockSpec((4, _BN, 128), lambda i: (0, i, 0)),
            pl.BlockSpec((_BN, 1), lambda i: (i, 0)),
            pl.BlockSpec((512, 512), lambda i: (0, 0)),
            pl.BlockSpec((1, 512), lambda i: (0, 0)),
            pl.BlockSpec((512, 128), lambda i: (0, 0)),
        ],
        out_specs=pl.BlockSpec((1, _BN, 128), lambda i: (0, i, 0)),
        out_shape=jax.ShapeDtypeStruct((1, _N, 128), jnp.float32),
    )(p2, g1, dinv, W2, b2r, W3)

    p3 = _agg_1(g3.reshape(_N, 128), srcb[:1], dstb, z8)

    out = pl.pallas_call(
        _t4_body,
        grid=(_GRID,),
        in_specs=[
            pl.BlockSpec((2, 1, _BN, 128), lambda i: (0, 0, i, 0)),
            pl.BlockSpec((1, _BN, 128), lambda i: (0, i, 0)),
            pl.BlockSpec((_BN, 1), lambda i: (i, 0)),
            pl.BlockSpec((1, 128), lambda i: (0, 0)),
        ],
        out_specs=pl.BlockSpec((_BN, 128), lambda i: (i, 0)),
        out_shape=jax.ShapeDtypeStruct((_N, 128), jnp.float32),
    )(p3, g3, dinv, b3r)

    return out
